# Initial kernel scaffold; baseline (speedup 1.0000x reference)
import jax
import jax.numpy as jnp
from jax.experimental import pallas as pl


def kernel(inputs, node_feature, edge_index, edge_type, edge_feature, W1, b1, SW1, sb1, EW1, eb1, W2, b2, SW2, sb2, EW2, eb2, M1W, M1b, M2W, M2b, F1W, F1b, F2W, F2b, F3W, F3b):
    raise NotImplementedError("write your pallas kernel here")


# speedup vs baseline: 4.2932x; 4.2932x over previous
import functools

import jax
import jax.numpy as jnp
from jax import lax
from jax.experimental import pallas as pl
from jax.experimental.pallas import tpu as pltpu
from jax.experimental.pallas import tpu_sc as plsc

N = 10000
E = 160000
R = 4
DN = 128
DE = 16
B = 4096
EPS = 1e-10

NC = 2
NS = 16
NW = NC * NS
CHUNK = 128
NCH = E // CHUNK
CPW = NCH // NW
EXTRA = NCH - CPW * NW
EPW = CPW * CHUNK
NBINS = N * R

_mesh = plsc.VectorSubcoreMesh(
    core_axis_name="c", subcore_axis_name="s", num_cores=NC, num_subcores=NS)

F32 = jnp.float32
I32 = jnp.int32


def _worker(c, s):
  wid = c * NS + s
  nmy = CPW + jnp.where(wid < EXTRA, 1, 0)
  cstart = wid * CPW + jnp.minimum(wid, EXTRA)
  return wid, nmy, cstart


def _load_edges(src, dst, typ, sv, dv, tv, wid, cstart):
  ebase = cstart * CHUNK
  pltpu.sync_copy(src.at[pl.ds(ebase, EPW)], sv.at[pl.ds(0, EPW)])
  pltpu.sync_copy(dst.at[pl.ds(ebase, EPW)], dv.at[pl.ds(0, EPW)])
  pltpu.sync_copy(typ.at[pl.ds(ebase, EPW)], tv.at[pl.ds(0, EPW)])

  @pl.when(wid < EXTRA)
  def _():
    pltpu.sync_copy(src.at[pl.ds(ebase + EPW, CHUNK)], sv.at[pl.ds(EPW, CHUNK)])
    pltpu.sync_copy(dst.at[pl.ds(ebase + EPW, CHUNK)], dv.at[pl.ds(EPW, CHUNK)])
    pltpu.sync_copy(typ.at[pl.ds(ebase + EPW, CHUNK)], tv.at[pl.ds(EPW, CHUNK)])


@functools.partial(
    pl.kernel,
    out_type=(
        jax.ShapeDtypeStruct((NC, NBINS, 32), F32),
        jax.ShapeDtypeStruct((NC, 4, R, N, 32), F32),
    ),
    mesh=_mesh,
    compiler_params=pltpu.CompilerParams(use_tc_tiling_on_sc=False),
    scratch_types=[
        pltpu.VMEM_SHARED((NBINS, 32), F32),
        pltpu.VMEM(((CPW + 1) * CHUNK,), I32),
        pltpu.VMEM(((CPW + 1) * CHUNK,), I32),
        pltpu.VMEM(((CPW + 1) * CHUNK,), I32),
        pltpu.VMEM((1, CHUNK), I32),
        pltpu.VMEM((CPW + 1, CHUNK), I32),
        pltpu.VMEM((CHUNK, 32), F32),
        pltpu.VMEM((CHUNK, DE), F32),
        pltpu.VMEM((CHUNK, 32), F32),
        pltpu.VMEM((80, 32), F32),
        pltpu.VMEM((80, 32), F32),
        pltpu.SemaphoreType.DMA,
    ],
)
def _sc1(x4, ef, src, dst, typ, out_e, out_x, acc, sv, dv, tv, gidx, sidx,
         rows, efv, aug, stage, zbuf, sem):
  c = lax.axis_index("c")
  s = lax.axis_index("s")
  wid, nmy, cstart = _worker(c, s)
  _load_edges(src, dst, typ, sv, dv, tv, wid, cstart)

  z16 = jnp.zeros((16,), F32)

  @pl.loop(0, 80)
  def _(i):
    zbuf[i, pl.ds(0, 16)] = z16
    zbuf[i, pl.ds(16, 16)] = z16

  nchk = 31 + jnp.where(s < 4, 1, 0)

  @pl.loop(0, nmy)
  def _(k):
    for i in range(8):
      off = k * CHUNK + i * 16
      d16 = dv[pl.ds(off, 16)]
      t16 = tv[pl.ds(off, 16)]
      sidx[k, pl.ds(i * 16, 16)] = t16 * N + d16

  def _zero_acc():
    @pl.loop(0, nchk)
    def _(j):
      pltpu.sync_copy(zbuf, acc.at[pl.ds((s + 16 * j) * 80, 80)])

  iota16 = lax.iota(I32, 16)
  one0 = jnp.where(iota16 == 0, 1.0, 0.0).astype(F32)

  @pl.loop(0, CHUNK)
  def _(rr):
    aug[rr, pl.ds(16, 16)] = one0

  _zero_acc()
  plsc.subcore_barrier()

  @pl.loop(0, nmy)
  def _(k):
    base = (cstart + k) * CHUNK
    pltpu.sync_copy(ef.at[pl.ds(base, CHUNK)], efv)

    @pl.loop(0, CHUNK)
    def _(rr):
      aug[rr, pl.ds(0, 16)] = efv[rr, pl.ds(0, 16)]

    pltpu.sync_copy(aug, acc.at[sidx.at[k]], add=True)

  plsc.subcore_barrier()

  @pl.loop(0, nchk)
  def _(j):
    off = (s + 16 * j) * 80
    pltpu.sync_copy(acc.at[pl.ds(off, 80)], stage)
    pltpu.sync_copy(stage, out_e.at[c, pl.ds(off, 80)])

  for q in range(4):
    plsc.subcore_barrier()
    _zero_acc()
    plsc.subcore_barrier()

    @pl.loop(0, nmy)
    def _(k):
      for i in range(8):
        off = k * CHUNK + i * 16
        s16 = sv[pl.ds(off, 16)]
        gidx[0, pl.ds(i * 16, 16)] = s16 * 4 + q
      pltpu.async_copy(x4.at[gidx.at[0]], rows, sem).wait()
      pltpu.sync_copy(rows, acc.at[sidx.at[k]], add=True)

    plsc.subcore_barrier()

    @pl.loop(0, nchk)
    def _(j):
      ch = s + 16 * j
      r_of = ch // 125
      node_off = (ch % 125) * 80
      pltpu.sync_copy(acc.at[pl.ds(ch * 80, 80)], stage)
      pltpu.sync_copy(stage, out_x.at[c, q, r_of, pl.ds(node_off, 80)])


@functools.partial(
    pl.kernel,
    out_type=(
        jax.ShapeDtypeStruct((NC, N, DN), F32),
        jax.ShapeDtypeStruct((B, DE), F32),
    ),
    mesh=_mesh,
    compiler_params=pltpu.CompilerParams(use_tc_tiling_on_sc=False),
    scratch_types=[
        pltpu.VMEM_SHARED((N, DN), F32),
        pltpu.VMEM(((CPW + 1) * CHUNK,), I32),
        pltpu.VMEM(((CPW + 1) * CHUNK,), I32),
        pltpu.VMEM(((CPW + 1) * CHUNK,), I32),
        pltpu.VMEM((1, CHUNK), I32),
        pltpu.VMEM((1, CHUNK), I32),
        pltpu.VMEM((1, CHUNK), I32),
        pltpu.VMEM((CHUNK, DN), F32),
        pltpu.VMEM((CHUNK, DE), F32),
        pltpu.VMEM((40, DN), F32),
        pltpu.VMEM((40, DN), F32),
        pltpu.VMEM((CHUNK,), I32),
        pltpu.VMEM((CHUNK, DE), F32),
        pltpu.SemaphoreType.DMA,
    ],
)
def _sc2(P4, winv, ef, src, dst, typ, cidx, out_s, ctx, acc, sv, dv, tv, gidx,
         widx, sidx, rows, wrows, stage, zbuf, cidxv, crows, sem):
  c = lax.axis_index("c")
  s = lax.axis_index("s")
  wid, nmy, cstart = _worker(c, s)
  _load_edges(src, dst, typ, sv, dv, tv, wid, cstart)

  z16 = jnp.zeros((16,), F32)

  @pl.loop(0, 40)
  def _(i):
    for k8 in range(8):
      zbuf[i, pl.ds(k8 * 16, 16)] = z16

  nchk = 15 + jnp.where(s < 10, 1, 0)

  @pl.loop(0, nchk)
  def _(j):
    pltpu.sync_copy(zbuf, acc.at[pl.ds((s + 16 * j) * 40, 40)])

  pltpu.sync_copy(cidx.at[pl.ds(wid * CHUNK, CHUNK)], cidxv)
  pltpu.async_copy(ef.at[cidxv], crows, sem).wait()
  pltpu.sync_copy(crows, ctx.at[pl.ds(wid * CHUNK, CHUNK)])

  plsc.subcore_barrier()

  @pl.loop(0, nmy)
  def _(k):
    for i in range(8):
      off = k * CHUNK + i * 16
      s16 = sv[pl.ds(off, 16)]
      d16 = dv[pl.ds(off, 16)]
      t16 = tv[pl.ds(off, 16)]
      gidx[0, pl.ds(i * 16, 16)] = s16 * 4 + t16
      widx[0, pl.ds(i * 16, 16)] = d16 * 4 + t16
      sidx[0, pl.ds(i * 16, 16)] = d16
    pltpu.async_copy(P4.at[gidx.at[0]], rows, sem).wait()
    pltpu.async_copy(winv.at[widx.at[0]], wrows, sem).wait()

    @pl.loop(0, 8)
    def _(g):
      for e in range(16):
        ridx = g * 16 + e
        wsp = wrows[ridx, pl.ds(0, 16)]
        for kk in range(8):
          rows[ridx, pl.ds(kk * 16, 16)] = rows[ridx, pl.ds(kk * 16, 16)] * wsp

    pltpu.sync_copy(rows, acc.at[sidx.at[0]], add=True)

  plsc.subcore_barrier()

  @pl.loop(0, nchk)
  def _(j):
    off = (s + 16 * j) * 40
    pltpu.sync_copy(acc.at[pl.ds(off, 40)], stage)
    pltpu.sync_copy(stage, out_s.at[c, pl.ds(off, 40)])


@functools.partial(
    pl.kernel,
    out_type=(
        jax.ShapeDtypeStruct((B, DN), F32),
        jax.ShapeDtypeStruct((B, DN), F32),
    ),
    mesh=_mesh,
    scratch_types=[
        pltpu.VMEM((CHUNK,), I32),
        pltpu.VMEM((CHUNK, DN), F32),
        pltpu.SemaphoreType.DMA,
    ],
)
def _sc3(h2, idx0, idx1, o1, o2, iv, rows, sem):
  c = lax.axis_index("c")
  s = lax.axis_index("s")
  wid = c * NS + s
  for idxref, oref in ((idx0, o1), (idx1, o2)):
    pltpu.sync_copy(idxref.at[pl.ds(wid * CHUNK, CHUNK)], iv)
    pltpu.async_copy(h2.at[iv], rows, sem).wait()
    pltpu.sync_copy(rows, oref.at[pl.ds(wid * CHUNK, CHUNK)])


_BN = 1000


def _tc1_body(x_ref, xp_ref, ep_ref, W1_ref, SW1_ref, bs1_ref, EW1_ref,
              eb1_ref, W2c_ref, SW2_ref, P_ref, sh2_ref, inv_ref, escat_ref,
              C_ref):
  f32 = jnp.float32
  xp = xp_ref[...]
  xs = xp[0] + xp[1]
  ep = ep_ref[...][0] + ep_ref[...][1]
  upd_parts, escat_parts, c_parts, inv_parts = [], [], [], []
  for r in range(R):
    esum_r = ep[r][:, 0:16]
    den_r = ep[r][:, 16:17]
    inv_r = 1.0 / (den_r + EPS)
    xsum_r = jnp.concatenate([xs[q, r] for q in range(4)], axis=-1)
    num_r = (xsum_r
             + jnp.dot(esum_r, EW1_ref[...], preferred_element_type=f32)
             + den_r * eb1_ref[...])
    upd_parts.append(num_r * inv_r)
    escat_parts.append(esum_r * inv_r)
    c_parts.append(den_r * inv_r)
    inv_parts.append(inv_r * jnp.ones((1, 16), f32))
  upd = jnp.concatenate(upd_parts, axis=-1)
  h1 = jnp.maximum(
      jnp.dot(upd, W1_ref[...], preferred_element_type=f32)
      + jnp.dot(x_ref[...], SW1_ref[...], preferred_element_type=f32)
      + bs1_ref[...], 0.0)
  P_ref[...] = jnp.dot(h1, W2c_ref[...], preferred_element_type=f32)
  sh2_ref[...] = jnp.dot(h1, SW2_ref[...], preferred_element_type=f32)
  inv_ref[...] = jnp.concatenate(inv_parts, axis=-1)
  escat_ref[...] = jnp.concatenate(escat_parts, axis=-1)
  C_ref[...] = jnp.concatenate(c_parts, axis=-1)


def _tc1(x, xp, ep, W1, SW1, bs1, EW1, eb1, W2c, SW2):
  full = lambda shape: pl.BlockSpec(shape, lambda i: (0,) * len(shape))
  return pl.pallas_call(
      _tc1_body,
      grid=(N // _BN,),
      compiler_params=pltpu.CompilerParams(
          vmem_limit_bytes=100 * 1024 * 1024),
      in_specs=[
          pl.BlockSpec((_BN, DN), lambda i: (i, 0)),
          pl.BlockSpec((NC, 4, R, _BN, 32), lambda i: (0, 0, 0, i, 0)),
          pl.BlockSpec((NC, R, _BN, 32), lambda i: (0, 0, i, 0)),
          full((512, 512)),
          full((DN, 512)),
          full((1, 512)),
          full((DE, DN)),
          full((1, DN)),
          full((512, 512)),
          full((512, DN)),
      ],
      out_specs=[
          pl.BlockSpec((_BN, 512), lambda i: (i, 0)),
          pl.BlockSpec((_BN, DN), lambda i: (i, 0)),
          pl.BlockSpec((_BN, R * 16), lambda i: (i, 0)),
          pl.BlockSpec((_BN, R * DE), lambda i: (i, 0)),
          pl.BlockSpec((_BN, R), lambda i: (i, 0)),
      ],
      out_shape=[
          jax.ShapeDtypeStruct((N, 512), F32),
          jax.ShapeDtypeStruct((N, DN), F32),
          jax.ShapeDtypeStruct((N, R * 16), F32),
          jax.ShapeDtypeStruct((N, R * DE), F32),
          jax.ShapeDtypeStruct((N, R), F32),
      ],
  )(x, xp, ep, W1, SW1, bs1, EW1, eb1, W2c, SW2)


def _tc2_body(sp_ref, escat_ref, C_ref, sh2_ref, EW2_ref, W2c_ref, eb2_ref,
              bs2_ref, h2_ref):
  f32 = jnp.float32
  sp = sp_ref[...]
  acc = sp[0] + sp[1] + sh2_ref[...] + bs2_ref[...]
  M = jnp.dot(EW2_ref[...], W2c_ref[...], preferred_element_type=f32)
  ebW = jnp.dot(eb2_ref[...], W2c_ref[...], preferred_element_type=f32)
  escat = escat_ref[...]
  C = C_ref[...]
  for r in range(R):
    acc = acc + jnp.dot(escat[:, 16 * r:16 * (r + 1)],
                        M[:, 128 * r:128 * (r + 1)],
                        preferred_element_type=f32)
    acc = acc + C[:, r:r + 1] * ebW[:, 128 * r:128 * (r + 1)]
  h2_ref[...] = jnp.maximum(acc, 0.0)


def _tc2(sp, escat, C, sh2, EW2, W2c, eb2, bs2):
  full = lambda shape: pl.BlockSpec(shape, lambda i: (0,) * len(shape))
  return pl.pallas_call(
      _tc2_body,
      grid=(N // _BN,),
      in_specs=[
          pl.BlockSpec((NC, _BN, DN), lambda i: (0, i, 0)),
          pl.BlockSpec((_BN, R * DE), lambda i: (i, 0)),
          pl.BlockSpec((_BN, R), lambda i: (i, 0)),
          pl.BlockSpec((_BN, DN), lambda i: (i, 0)),
          full((DE, 512)),
          full((512, 512)),
          full((1, 512)),
          full((1, DN)),
      ],
      out_specs=pl.BlockSpec((_BN, DN), lambda i: (i, 0)),
      out_shape=jax.ShapeDtypeStruct((N, DN), F32),
  )(sp, escat, C, sh2, EW2, W2c, eb2, bs2)


_BB = 1024


def _tc3_body(x1_ref, x2_ref, ctx_ref, M1W_ref, M1b_ref, M2W_ref, M2b_ref,
              F1W_ref, F1b_ref, F2W_ref, F2b_ref, F3W_ref, F3b_ref, o_ref):
  f32 = jnp.float32
  m = jnp.maximum(
      jnp.dot(ctx_ref[...], M1W_ref[...], preferred_element_type=f32)
      + M1b_ref[...], 0.0)
  m = jnp.dot(m, M2W_ref[...], preferred_element_type=f32) + M2b_ref[...]
  F1W = F1W_ref[...]
  h = jnp.maximum(
      jnp.dot(x1_ref[...], F1W[0:128], preferred_element_type=f32)
      + jnp.dot(x2_ref[...], F1W[128:256], preferred_element_type=f32)
      + jnp.dot(m, F1W[256:384], preferred_element_type=f32)
      + F1b_ref[...], 0.0)
  h = jnp.maximum(
      jnp.dot(h, F2W_ref[...], preferred_element_type=f32) + F2b_ref[...], 0.0)
  o_ref[...] = jnp.dot(h, F3W_ref[...], preferred_element_type=f32) + F3b_ref[...]


def _tc3(x1, x2, ctx, M1W, M1b, M2W, M2b, F1W, F1b, F2W, F2b, F3W, F3b):
  full = lambda shape: pl.BlockSpec(shape, lambda i: (0,) * len(shape))
  return pl.pallas_call(
      _tc3_body,
      grid=(B // _BB,),
      in_specs=[
          pl.BlockSpec((_BB, DN), lambda i: (i, 0)),
          pl.BlockSpec((_BB, DN), lambda i: (i, 0)),
          pl.BlockSpec((_BB, DE), lambda i: (i, 0)),
          full((DE, 256)),
          full((1, 256)),
          full((256, DN)),
          full((1, DN)),
          full((384, DN)),
          full((1, DN)),
          full((DN, 64)),
          full((1, 64)),
          full((64, 1)),
          full((1, 1)),
      ],
      out_specs=pl.BlockSpec((_BB, 1), lambda i: (i, 0)),
      out_shape=jax.ShapeDtypeStruct((B, 1), F32),
  )(x1, x2, ctx, M1W, M1b, M2W, M2b, F1W, F1b, F2W, F2b, F3W, F3b)


def kernel(inputs, node_feature, edge_index, edge_type, edge_feature,
           W1, b1, SW1, sb1, EW1, eb1,
           W2, b2, SW2, sb2, EW2, eb2,
           M1W, M1b, M2W, M2b,
           F1W, F1b, F2W, F2b, F3W, F3b):
  src = edge_index[0]
  dst = edge_index[1]
  x4 = node_feature.reshape(N * 4, 32)

  out_e, out_x = _sc1(x4, edge_feature, src, dst, edge_type)

  W2cat = W2.reshape(R, 512, DN).transpose(1, 0, 2).reshape(512, R * DN)
  P, sh2, inv, escat, C = _tc1(
      node_feature, out_x, out_e.reshape(NC, R, N, 32), W1, SW1,
      (b1 + sb1).reshape(1, 512),
      EW1, eb1.reshape(1, DN), W2cat, SW2)

  P4 = P.reshape(N * R, DN)
  winv = inv.reshape(N * R, 16)
  out_s, ctx = _sc2(P4, winv, edge_feature, src, dst, edge_type, inputs[:, 2])

  h2 = _tc2(out_s, escat, C, sh2, EW2, W2cat, eb2.reshape(1, 512),
            (b2 + sb2).reshape(1, DN))

  x1, x2 = _sc3(h2, inputs[:, 0], inputs[:, 1])

  return _tc3(x1, x2, ctx, M1W, M1b.reshape(1, 256), M2W, M2b.reshape(1, DN),
              F1W, F1b.reshape(1, DN), F2W, F2b.reshape(1, 64), F3W,
              F3b.reshape(1, 1))


# speedup vs baseline: 4.7991x; 1.1178x over previous
import functools

import jax
import jax.numpy as jnp
from jax import lax
from jax.experimental import pallas as pl
from jax.experimental.pallas import tpu as pltpu
from jax.experimental.pallas import tpu_sc as plsc

N = 10000
E = 160000
R = 4
DN = 128
DE = 16
B = 4096
EPS = 1e-10

NC = 2
NS = 16
NW = NC * NS
CHUNK = 128
NCH = E // CHUNK
CPW = NCH // NW
EXTRA = NCH - CPW * NW
EPW = CPW * CHUNK
NBINS = N * R

_mesh = plsc.VectorSubcoreMesh(
    core_axis_name="c", subcore_axis_name="s", num_cores=NC, num_subcores=NS)

F32 = jnp.float32
I32 = jnp.int32


def _worker(c, s):
  wid = c * NS + s
  nmy = CPW + jnp.where(wid < EXTRA, 1, 0)
  cstart = wid * CPW + jnp.minimum(wid, EXTRA)
  return wid, nmy, cstart


def _load_edges(src, dst, typ, sv, dv, tv, wid, cstart):
  ebase = cstart * CHUNK
  pltpu.sync_copy(src.at[pl.ds(ebase, EPW)], sv.at[pl.ds(0, EPW)])
  pltpu.sync_copy(dst.at[pl.ds(ebase, EPW)], dv.at[pl.ds(0, EPW)])
  pltpu.sync_copy(typ.at[pl.ds(ebase, EPW)], tv.at[pl.ds(0, EPW)])

  @pl.when(wid < EXTRA)
  def _():
    pltpu.sync_copy(src.at[pl.ds(ebase + EPW, CHUNK)], sv.at[pl.ds(EPW, CHUNK)])
    pltpu.sync_copy(dst.at[pl.ds(ebase + EPW, CHUNK)], dv.at[pl.ds(EPW, CHUNK)])
    pltpu.sync_copy(typ.at[pl.ds(ebase + EPW, CHUNK)], tv.at[pl.ds(EPW, CHUNK)])


@functools.partial(
    pl.kernel,
    out_type=(
        jax.ShapeDtypeStruct((NC, NBINS, 32), F32),
        jax.ShapeDtypeStruct((NC, 4, R, N, 32), F32),
    ),
    mesh=_mesh,
    compiler_params=pltpu.CompilerParams(use_tc_tiling_on_sc=False),
    scratch_types=[
        pltpu.VMEM_SHARED((NBINS, 32), F32),
        pltpu.VMEM(((CPW + 1) * CHUNK,), I32),
        pltpu.VMEM(((CPW + 1) * CHUNK,), I32),
        pltpu.VMEM(((CPW + 1) * CHUNK,), I32),
        pltpu.VMEM((1, CHUNK), I32),
        pltpu.VMEM((1, CHUNK), I32),
        pltpu.VMEM((CPW + 1, CHUNK), I32),
        pltpu.VMEM((CHUNK, 32), F32),
        pltpu.VMEM((CHUNK, 32), F32),
        pltpu.VMEM((CHUNK, DE), F32),
        pltpu.VMEM((CHUNK, 32), F32),
        pltpu.VMEM((80, 32), F32),
        pltpu.VMEM((80, 32), F32),
        pltpu.SemaphoreType.DMA,
        pltpu.SemaphoreType.DMA,
    ],
)
def _sc1(x4, ef, src, dst, typ, out_e, out_x, acc, sv, dv, tv, gidx, gidx2,
         sidx, rows, rows2, efv, aug, stage, zbuf, sem, sem2):
  c = lax.axis_index("c")
  s = lax.axis_index("s")
  wid, nmy, cstart = _worker(c, s)
  _load_edges(src, dst, typ, sv, dv, tv, wid, cstart)

  z16 = jnp.zeros((16,), F32)

  @pl.loop(0, 80)
  def _(i):
    zbuf[i, pl.ds(0, 16)] = z16
    zbuf[i, pl.ds(16, 16)] = z16

  nchk = 31 + jnp.where(s < 4, 1, 0)

  @pl.loop(0, nmy)
  def _(k):
    for i in range(8):
      off = k * CHUNK + i * 16
      d16 = dv[pl.ds(off, 16)]
      t16 = tv[pl.ds(off, 16)]
      sidx[k, pl.ds(i * 16, 16)] = t16 * N + d16

  def _zero_acc():
    @pl.loop(0, nchk)
    def _(j):
      pltpu.sync_copy(zbuf, acc.at[pl.ds((s + 16 * j) * 80, 80)])

  iota16 = lax.iota(I32, 16)
  one0 = jnp.where(iota16 == 0, 1.0, 0.0).astype(F32)

  @pl.loop(0, CHUNK)
  def _(rr):
    aug[rr, pl.ds(16, 16)] = one0

  _zero_acc()
  plsc.subcore_barrier()

  @pl.loop(0, nmy)
  def _(k):
    base = (cstart + k) * CHUNK
    pltpu.sync_copy(ef.at[pl.ds(base, CHUNK)], efv)

    @pl.loop(0, CHUNK)
    def _(rr):
      aug[rr, pl.ds(0, 16)] = efv[rr, pl.ds(0, 16)]

    pltpu.sync_copy(aug, acc.at[sidx.at[k]], add=True)

  plsc.subcore_barrier()

  @pl.loop(0, nchk)
  def _(j):
    off = (s + 16 * j) * 80
    pltpu.sync_copy(acc.at[pl.ds(off, 80)], stage)
    pltpu.sync_copy(stage, out_e.at[c, pl.ds(off, 80)])

  for q in range(4):
    plsc.subcore_barrier()
    _zero_acc()
    plsc.subcore_barrier()

    def _fire(k, gbuf, rbuf, sm):
      for i in range(8):
        off = k * CHUNK + i * 16
        s16 = sv[pl.ds(off, 16)]
        gbuf[0, pl.ds(i * 16, 16)] = s16 * 4 + q
      return pltpu.async_copy(x4.at[gbuf.at[0]], rbuf, sm)

    npair = nmy // 2

    @pl.loop(0, npair)
    def _(p):
      d0 = _fire(2 * p, gidx, rows, sem)
      d1 = _fire(2 * p + 1, gidx2, rows2, sem2)
      d0.wait()
      pltpu.sync_copy(rows, acc.at[sidx.at[2 * p]], add=True)
      d1.wait()
      pltpu.sync_copy(rows2, acc.at[sidx.at[2 * p + 1]], add=True)

    @pl.when(nmy % 2 == 1)
    def _():
      k = nmy - 1
      _fire(k, gidx, rows, sem).wait()
      pltpu.sync_copy(rows, acc.at[sidx.at[k]], add=True)

    plsc.subcore_barrier()

    @pl.loop(0, nchk)
    def _(j):
      ch = s + 16 * j
      r_of = ch // 125
      node_off = (ch % 125) * 80
      pltpu.sync_copy(acc.at[pl.ds(ch * 80, 80)], stage)
      pltpu.sync_copy(stage, out_x.at[c, q, r_of, pl.ds(node_off, 80)])


@functools.partial(
    pl.kernel,
    out_type=(
        jax.ShapeDtypeStruct((NC, N, DN), F32),
        jax.ShapeDtypeStruct((B, DE), F32),
    ),
    mesh=_mesh,
    compiler_params=pltpu.CompilerParams(use_tc_tiling_on_sc=False),
    scratch_types=[
        pltpu.VMEM_SHARED((N, DN), F32),
        pltpu.VMEM(((CPW + 1) * CHUNK,), I32),
        pltpu.VMEM(((CPW + 1) * CHUNK,), I32),
        pltpu.VMEM(((CPW + 1) * CHUNK,), I32),
        pltpu.VMEM((1, 64), I32),
        pltpu.VMEM((1, 64), I32),
        pltpu.VMEM((1, 64), I32),
        pltpu.VMEM((1, 64), I32),
        pltpu.VMEM((1, 64), I32),
        pltpu.VMEM((1, 64), I32),
        pltpu.VMEM((64, DN), F32),
        pltpu.VMEM((64, DN), F32),
        pltpu.VMEM((64, DE), F32),
        pltpu.VMEM((64, DE), F32),
        pltpu.VMEM((40, DN), F32),
        pltpu.VMEM((40, DN), F32),
        pltpu.VMEM((CHUNK,), I32),
        pltpu.VMEM((CHUNK, DE), F32),
        pltpu.SemaphoreType.DMA,
        pltpu.SemaphoreType.DMA,
    ],
)
def _sc2(P4, winv, ef, src, dst, typ, cidx, out_s, ctx, acc, sv, dv, tv,
         gidx, gidx2, widx, widx2, sidx, sidx2, rows, rows2, wrows, wrows2,
         stage, zbuf, cidxv, crows, sem, sem2):
  c = lax.axis_index("c")
  s = lax.axis_index("s")
  wid, nmy, cstart = _worker(c, s)
  _load_edges(src, dst, typ, sv, dv, tv, wid, cstart)

  z16 = jnp.zeros((16,), F32)

  @pl.loop(0, 40)
  def _(i):
    for k8 in range(8):
      zbuf[i, pl.ds(k8 * 16, 16)] = z16

  nchk = 15 + jnp.where(s < 10, 1, 0)

  @pl.loop(0, nchk)
  def _(j):
    pltpu.sync_copy(zbuf, acc.at[pl.ds((s + 16 * j) * 40, 40)])

  pltpu.sync_copy(cidx.at[pl.ds(wid * CHUNK, CHUNK)], cidxv)
  pltpu.async_copy(ef.at[cidxv], crows, sem).wait()
  pltpu.sync_copy(crows, ctx.at[pl.ds(wid * CHUNK, CHUNK)])

  plsc.subcore_barrier()

  def _fire(h, gbuf, wbuf, sbuf, rbuf, wrbuf, sm):
    for i in range(4):
      off = h * 64 + i * 16
      s16 = sv[pl.ds(off, 16)]
      d16 = dv[pl.ds(off, 16)]
      t16 = tv[pl.ds(off, 16)]
      gbuf[0, pl.ds(i * 16, 16)] = s16 * 4 + t16
      wbuf[0, pl.ds(i * 16, 16)] = d16 * 4 + t16
      sbuf[0, pl.ds(i * 16, 16)] = d16
    d_a = pltpu.async_copy(P4.at[gbuf.at[0]], rbuf, sm)
    d_b = pltpu.async_copy(winv.at[wbuf.at[0]], wrbuf, sm)
    return d_a, d_b

  def _weight_scatter(rbuf, wrbuf, sbuf):
    @pl.loop(0, 4)
    def _(g):
      for e in range(16):
        ridx = g * 16 + e
        wsp = wrbuf[ridx, pl.ds(0, 16)]
        for kk in range(8):
          rbuf[ridx, pl.ds(kk * 16, 16)] = (
              rbuf[ridx, pl.ds(kk * 16, 16)] * wsp)
    pltpu.sync_copy(rbuf, acc.at[sbuf.at[0]], add=True)

  nsub = nmy * 2

  @pl.loop(0, nsub // 2)
  def _(p):
    da0, db0 = _fire(2 * p, gidx, widx, sidx, rows, wrows, sem)
    da1, db1 = _fire(2 * p + 1, gidx2, widx2, sidx2, rows2, wrows2, sem2)
    da0.wait()
    db0.wait()
    _weight_scatter(rows, wrows, sidx)
    da1.wait()
    db1.wait()
    _weight_scatter(rows2, wrows2, sidx2)

  plsc.subcore_barrier()

  @pl.loop(0, nchk)
  def _(j):
    off = (s + 16 * j) * 40
    pltpu.sync_copy(acc.at[pl.ds(off, 40)], stage)
    pltpu.sync_copy(stage, out_s.at[c, pl.ds(off, 40)])


@functools.partial(
    pl.kernel,
    out_type=(
        jax.ShapeDtypeStruct((B, DN), F32),
        jax.ShapeDtypeStruct((B, DN), F32),
    ),
    mesh=_mesh,
    scratch_types=[
        pltpu.VMEM((CHUNK,), I32),
        pltpu.VMEM((CHUNK, DN), F32),
        pltpu.SemaphoreType.DMA,
    ],
)
def _sc3(h2, idx0, idx1, o1, o2, iv, rows, sem):
  c = lax.axis_index("c")
  s = lax.axis_index("s")
  wid = c * NS + s
  for idxref, oref in ((idx0, o1), (idx1, o2)):
    pltpu.sync_copy(idxref.at[pl.ds(wid * CHUNK, CHUNK)], iv)
    pltpu.async_copy(h2.at[iv], rows, sem).wait()
    pltpu.sync_copy(rows, oref.at[pl.ds(wid * CHUNK, CHUNK)])


_BN = 1000


def _tc1_body(x_ref, xp_ref, ep_ref, W1_ref, SW1_ref, bs1_ref, EW1_ref,
              eb1_ref, W2c_ref, SW2_ref, P_ref, sh2_ref, inv_ref, escat_ref,
              C_ref):
  f32 = jnp.float32
  xp = xp_ref[...]
  xs = xp[0] + xp[1]
  ep = ep_ref[...][0] + ep_ref[...][1]
  upd_parts, escat_parts, c_parts, inv_parts = [], [], [], []
  for r in range(R):
    esum_r = ep[r][:, 0:16]
    den_r = ep[r][:, 16:17]
    inv_r = 1.0 / (den_r + EPS)
    xsum_r = jnp.concatenate([xs[q, r] for q in range(4)], axis=-1)
    num_r = (xsum_r
             + jnp.dot(esum_r, EW1_ref[...], preferred_element_type=f32)
             + den_r * eb1_ref[...])
    upd_parts.append(num_r * inv_r)
    escat_parts.append(esum_r * inv_r)
    c_parts.append(den_r * inv_r)
    inv_parts.append(inv_r * jnp.ones((1, 16), f32))
  upd = jnp.concatenate(upd_parts, axis=-1)
  h1 = jnp.maximum(
      jnp.dot(upd, W1_ref[...], preferred_element_type=f32)
      + jnp.dot(x_ref[...], SW1_ref[...], preferred_element_type=f32)
      + bs1_ref[...], 0.0)
  P_ref[...] = jnp.dot(h1, W2c_ref[...], preferred_element_type=f32)
  sh2_ref[...] = jnp.dot(h1, SW2_ref[...], preferred_element_type=f32)
  inv_ref[...] = jnp.concatenate(inv_parts, axis=-1)
  escat_ref[...] = jnp.concatenate(escat_parts, axis=-1)
  C_ref[...] = jnp.concatenate(c_parts, axis=-1)


def _tc1(x, xp, ep, W1, SW1, bs1, EW1, eb1, W2c, SW2):
  full = lambda shape: pl.BlockSpec(shape, lambda i: (0,) * len(shape))
  return pl.pallas_call(
      _tc1_body,
      grid=(N // _BN,),
      compiler_params=pltpu.CompilerParams(
          vmem_limit_bytes=100 * 1024 * 1024),
      in_specs=[
          pl.BlockSpec((_BN, DN), lambda i: (i, 0)),
          pl.BlockSpec((NC, 4, R, _BN, 32), lambda i: (0, 0, 0, i, 0)),
          pl.BlockSpec((NC, R, _BN, 32), lambda i: (0, 0, i, 0)),
          full((512, 512)),
          full((DN, 512)),
          full((1, 512)),
          full((DE, DN)),
          full((1, DN)),
          full((512, 512)),
          full((512, DN)),
      ],
      out_specs=[
          pl.BlockSpec((_BN, 512), lambda i: (i, 0)),
          pl.BlockSpec((_BN, DN), lambda i: (i, 0)),
          pl.BlockSpec((_BN, R * 16), lambda i: (i, 0)),
          pl.BlockSpec((_BN, R * DE), lambda i: (i, 0)),
          pl.BlockSpec((_BN, R), lambda i: (i, 0)),
      ],
      out_shape=[
          jax.ShapeDtypeStruct((N, 512), F32),
          jax.ShapeDtypeStruct((N, DN), F32),
          jax.ShapeDtypeStruct((N, R * 16), F32),
          jax.ShapeDtypeStruct((N, R * DE), F32),
          jax.ShapeDtypeStruct((N, R), F32),
      ],
  )(x, xp, ep, W1, SW1, bs1, EW1, eb1, W2c, SW2)


def _tc2_body(sp_ref, escat_ref, C_ref, sh2_ref, EW2_ref, W2c_ref, eb2_ref,
              bs2_ref, h2_ref):
  f32 = jnp.float32
  sp = sp_ref[...]
  acc = sp[0] + sp[1] + sh2_ref[...] + bs2_ref[...]
  M = jnp.dot(EW2_ref[...], W2c_ref[...], preferred_element_type=f32)
  ebW = jnp.dot(eb2_ref[...], W2c_ref[...], preferred_element_type=f32)
  escat = escat_ref[...]
  C = C_ref[...]
  for r in range(R):
    acc = acc + jnp.dot(escat[:, 16 * r:16 * (r + 1)],
                        M[:, 128 * r:128 * (r + 1)],
                        preferred_element_type=f32)
    acc = acc + C[:, r:r + 1] * ebW[:, 128 * r:128 * (r + 1)]
  h2_ref[...] = jnp.maximum(acc, 0.0)


def _tc2(sp, escat, C, sh2, EW2, W2c, eb2, bs2):
  full = lambda shape: pl.BlockSpec(shape, lambda i: (0,) * len(shape))
  return pl.pallas_call(
      _tc2_body,
      grid=(N // _BN,),
      in_specs=[
          pl.BlockSpec((NC, _BN, DN), lambda i: (0, i, 0)),
          pl.BlockSpec((_BN, R * DE), lambda i: (i, 0)),
          pl.BlockSpec((_BN, R), lambda i: (i, 0)),
          pl.BlockSpec((_BN, DN), lambda i: (i, 0)),
          full((DE, 512)),
          full((512, 512)),
          full((1, 512)),
          full((1, DN)),
      ],
      out_specs=pl.BlockSpec((_BN, DN), lambda i: (i, 0)),
      out_shape=jax.ShapeDtypeStruct((N, DN), F32),
  )(sp, escat, C, sh2, EW2, W2c, eb2, bs2)


_BB = 1024


def _tc3_body(x1_ref, x2_ref, ctx_ref, M1W_ref, M1b_ref, M2W_ref, M2b_ref,
              F1W_ref, F1b_ref, F2W_ref, F2b_ref, F3W_ref, F3b_ref, o_ref):
  f32 = jnp.float32
  m = jnp.maximum(
      jnp.dot(ctx_ref[...], M1W_ref[...], preferred_element_type=f32)
      + M1b_ref[...], 0.0)
  m = jnp.dot(m, M2W_ref[...], preferred_element_type=f32) + M2b_ref[...]
  F1W = F1W_ref[...]
  h = jnp.maximum(
      jnp.dot(x1_ref[...], F1W[0:128], preferred_element_type=f32)
      + jnp.dot(x2_ref[...], F1W[128:256], preferred_element_type=f32)
      + jnp.dot(m, F1W[256:384], preferred_element_type=f32)
      + F1b_ref[...], 0.0)
  h = jnp.maximum(
      jnp.dot(h, F2W_ref[...], preferred_element_type=f32) + F2b_ref[...], 0.0)
  o_ref[...] = jnp.dot(h, F3W_ref[...], preferred_element_type=f32) + F3b_ref[...]


def _tc3(x1, x2, ctx, M1W, M1b, M2W, M2b, F1W, F1b, F2W, F2b, F3W, F3b):
  full = lambda shape: pl.BlockSpec(shape, lambda i: (0,) * len(shape))
  return pl.pallas_call(
      _tc3_body,
      grid=(B // _BB,),
      in_specs=[
          pl.BlockSpec((_BB, DN), lambda i: (i, 0)),
          pl.BlockSpec((_BB, DN), lambda i: (i, 0)),
          pl.BlockSpec((_BB, DE), lambda i: (i, 0)),
          full((DE, 256)),
          full((1, 256)),
          full((256, DN)),
          full((1, DN)),
          full((384, DN)),
          full((1, DN)),
          full((DN, 64)),
          full((1, 64)),
          full((64, 1)),
          full((1, 1)),
      ],
      out_specs=pl.BlockSpec((_BB, 1), lambda i: (i, 0)),
      out_shape=jax.ShapeDtypeStruct((B, 1), F32),
  )(x1, x2, ctx, M1W, M1b, M2W, M2b, F1W, F1b, F2W, F2b, F3W, F3b)


def kernel(inputs, node_feature, edge_index, edge_type, edge_feature,
           W1, b1, SW1, sb1, EW1, eb1,
           W2, b2, SW2, sb2, EW2, eb2,
           M1W, M1b, M2W, M2b,
           F1W, F1b, F2W, F2b, F3W, F3b):
  src = edge_index[0]
  dst = edge_index[1]
  x4 = node_feature.reshape(N * 4, 32)

  out_e, out_x = _sc1(x4, edge_feature, src, dst, edge_type)

  W2cat = W2.reshape(R, 512, DN).transpose(1, 0, 2).reshape(512, R * DN)
  P, sh2, inv, escat, C = _tc1(
      node_feature, out_x, out_e.reshape(NC, R, N, 32), W1, SW1,
      (b1 + sb1).reshape(1, 512),
      EW1, eb1.reshape(1, DN), W2cat, SW2)

  P4 = P.reshape(N * R, DN)
  winv = inv.reshape(N * R, 16)
  out_s, ctx = _sc2(P4, winv, edge_feature, src, dst, edge_type, inputs[:, 2])

  h2 = _tc2(out_s, escat, C, sh2, EW2, W2cat, eb2.reshape(1, 512),
            (b2 + sb2).reshape(1, DN))

  x1, x2 = _sc3(h2, inputs[:, 0], inputs[:, 1])

  return _tc3(x1, x2, ctx, M1W, M1b.reshape(1, 256), M2W, M2b.reshape(1, DN),
              F1W, F1b.reshape(1, DN), F2W, F2b.reshape(1, 64), F3W,
              F3b.reshape(1, 1))
